# baseline (speedup 1.0000x reference)
import functools

import jax
import jax.numpy as jnp
from jax.experimental import pallas as pl
from jax.experimental.pallas import tpu as pltpu
from jax.experimental.pallas import tpu_sc as plsc

L = 8192
F = L // 2 + 1
K = 4

NCLS = 8
NJ = L // NCLS
FH = NJ // 2
BF = 264
NFB = 2
NCAND = 8 * K
KBJ = 256
NKB = NJ // KBJ
LC = 1024
NL0 = L // LC

_PI = 3.14159265358979323846


def _dft_topk_kernel(x_ref, val_ref, f_ref, re_ref, im_ref, acc):
    fb = pl.program_id(1)
    kb = pl.program_id(2)
    f0 = fb * BF
    j0 = kb * KBJ

    frow = jax.lax.broadcasted_iota(jnp.int32, (BF, KBJ), 0) + f0
    jcol = jax.lax.broadcasted_iota(jnp.int32, (BF, KBJ), 1) + j0
    m = (frow * jcol) & (NJ - 1)
    ang = m.astype(jnp.float32) * (2.0 * _PI / NJ)
    bas = jnp.concatenate([jnp.cos(ang), -jnp.sin(ang)], axis=0)

    for c in range(NCLS):
        xc = x_ref[0, :, c, :]
        p = jnp.dot(bas, xc, preferred_element_type=jnp.float32,
                    precision=jax.lax.Precision.HIGHEST)

        @pl.when(kb == 0)
        def _():
            acc[c] = p

        @pl.when(kb > 0)
        def _():
            acc[c] += p

    @pl.when(kb == NKB - 1)
    def _():
        d = x_ref.shape[3]
        fcol = jax.lax.broadcasted_iota(jnp.int32, (BF, 1), 0) + f0
        rowid = jax.lax.broadcasted_iota(jnp.int32, (BF, d), 0)
        inrange = fcol <= FH
        sub = 0
        for q in range(5):
            for sgnform in (0, 1):
                if sgnform == 0 and q == 4:
                    continue
                if sgnform == 1 and q == 0:
                    continue
                conj = sgnform == 1
                fm = NJ * q + fcol if not conj else NJ * q - fcol
                sgn = -1.0 if conj else 1.0
                rs = acc[0, 0:BF, :]
                is_ = acc[0, BF:2 * BF, :] * sgn
                for c in range(1, NCLS):
                    angt = (((fm * c) & (L - 1)).astype(jnp.float32)
                            * (2.0 * _PI / L))
                    tre = jnp.cos(angt)
                    tim = -jnp.sin(angt)
                    gre = acc[c, 0:BF, :]
                    gim = acc[c, BF:2 * BF, :] * sgn
                    rs = rs + tre * gre - tim * gim
                    is_ = is_ + tre * gim + tim * gre
                valid = inrange
                if conj:
                    if q < 4:
                        valid = valid & (fcol != 0)
                    valid = valid & (fcol != FH)
                amp2 = rs * rs + is_ * is_
                work = jnp.where(valid, amp2, -1.0)
                fmf = jnp.broadcast_to(fm, (BF, d)).astype(jnp.float32)
                for r in range(K):
                    mx = jnp.max(work, axis=0)
                    cand = jnp.where(work == mx[None, :], rowid,
                                     jnp.int32(2**30))
                    idx = jnp.min(cand, axis=0)
                    oh = rowid == idx[None, :]
                    slot = sub * K + r
                    val_ref[0, 0, slot, :] = mx
                    f_ref[0, 0, slot, :] = jnp.sum(
                        jnp.where(oh, fmf, 0.0), axis=0)
                    re_ref[0, 0, slot, :] = jnp.sum(
                        jnp.where(oh, rs, 0.0), axis=0)
                    im_ref[0, 0, slot, :] = jnp.sum(
                        jnp.where(oh, is_, 0.0), axis=0)
                    work = jnp.where(oh, -1.0, work)
                sub += 1


def _sc_merge_builder(b, d):
    mesh = plsc.VectorSubcoreMesh(core_axis_name="c", subcore_axis_name="s")
    info = plsc.get_sparse_core_info()
    slabs_per_b = d // 128

    @functools.partial(
        pl.kernel, mesh=mesh,
        out_type=[jax.ShapeDtypeStruct((b, K, d), jnp.float32)] * 3,
        scratch_types=[pltpu.VMEM((NFB, NCAND, 128), jnp.float32)] * 4
                      + [pltpu.VMEM((K, 128), jnp.float32)] * 3,
    )
    def sc_merge(val_h, f_h, re_h, im_h, fo_h, ao_h, bo_h,
                 vs, fs, rs, is2, fo_s, ao_s, bo_s):
        wid = jax.lax.axis_index("s") * info.num_cores + jax.lax.axis_index("c")
        bi = wid // slabs_per_b
        d0 = (wid % slabs_per_b) * 128
        pltpu.sync_copy(val_h.at[bi, :, :, pl.ds(d0, 128)], vs)
        pltpu.sync_copy(f_h.at[bi, :, :, pl.ds(d0, 128)], fs)
        pltpu.sync_copy(re_h.at[bi, :, :, pl.ds(d0, 128)], rs)
        pltpu.sync_copy(im_h.at[bi, :, :, pl.ds(d0, 128)], is2)

        def body(dj, carry):
            j0 = dj * 16
            tv = [jnp.full((16,), -2.0, jnp.float32) for _ in range(K)]
            tf = [jnp.zeros((16,), jnp.float32) for _ in range(K)]
            tr = [jnp.zeros((16,), jnp.float32) for _ in range(K)]
            ti = [jnp.zeros((16,), jnp.float32) for _ in range(K)]
            for fb in range(NFB):
                for s in range(NCAND):
                    v = vs[fb, s, pl.ds(j0, 16)]
                    fv = fs[fb, s, pl.ds(j0, 16)]
                    rv = rs[fb, s, pl.ds(j0, 16)]
                    iv = is2[fb, s, pl.ds(j0, 16)]
                    for k in range(K):
                        gt = v > tv[k]
                        tv[k], v = (jnp.where(gt, v, tv[k]),
                                    jnp.where(gt, tv[k], v))
                        tf[k], fv = (jnp.where(gt, fv, tf[k]),
                                     jnp.where(gt, tf[k], fv))
                        tr[k], rv = (jnp.where(gt, rv, tr[k]),
                                     jnp.where(gt, tr[k], rv))
                        ti[k], iv = (jnp.where(gt, iv, ti[k]),
                                     jnp.where(gt, ti[k], iv))
            for k in range(K):
                w = jnp.where((tf[k] == 0.0) | (tf[k] == float(L // 2)),
                              1.0 / L, 2.0 / L)
                fo_s[k, pl.ds(j0, 16)] = tf[k]
                ao_s[k, pl.ds(j0, 16)] = w * tr[k]
                bo_s[k, pl.ds(j0, 16)] = -(w * ti[k])
            return carry

        jax.lax.fori_loop(0, 8, body, 0)
        pltpu.sync_copy(fo_s, fo_h.at[bi, :, pl.ds(d0, 128)])
        pltpu.sync_copy(ao_s, ao_h.at[bi, :, pl.ds(d0, 128)])
        pltpu.sync_copy(bo_s, bo_h.at[bi, :, pl.ds(d0, 128)])

    return sc_merge


def _recon_kernel(fin_ref, ain_ref, bin_ref, out_ref,
                  fsel, asel, bsel, cosA, sinA):
    l0 = pl.program_id(1)
    d = out_ref.shape[2]

    @pl.when(l0 == 0)
    def _():
        for r in range(K):
            fsel[r, :] = fin_ref[0, r, :].astype(jnp.int32)
            asel[r, :] = ain_ref[0, r, :]
            bsel[r, :] = bin_ref[0, r, :]
        for r in range(K):
            fi = fsel[r, :][None, :]
            dl = jax.lax.broadcasted_iota(jnp.int32, (LC, d), 0)
            mm = (dl * fi) & (L - 1)
            a = mm.astype(jnp.float32) * (2.0 * _PI / L)
            cosA[r] = jnp.cos(a)
            sinA[r] = jnp.sin(a)

    acc = jnp.zeros((LC, d), jnp.float32)
    for r in range(K):
        fi = fsel[r, :]
        ph = (fi * (LC * l0)) & (L - 1)
        a0 = ph.astype(jnp.float32) * (2.0 * _PI / L)
        cp = jnp.cos(a0)[None, :]
        sp = jnp.sin(a0)[None, :]
        A = asel[r, :][None, :]
        Bc = bsel[r, :][None, :]
        U = A * cp + Bc * sp
        V = Bc * cp - A * sp
        acc = acc + U * cosA[r] + V * sinA[r]
    out_ref[0] = acc


def kernel(x):
    b, l, d = x.shape
    assert l == L
    xr = x.reshape(b, NJ, NCLS, d)

    cand_shape = jax.ShapeDtypeStruct((b, NFB, NCAND, d), jnp.float32)
    val, fsel, re, im = pl.pallas_call(
        _dft_topk_kernel,
        grid=(b, NFB, NKB),
        in_specs=[pl.BlockSpec((1, KBJ, NCLS, d),
                               lambda bi, fb, kb: (bi, kb, 0, 0))],
        out_specs=[pl.BlockSpec((1, 1, NCAND, d),
                                lambda bi, fb, kb: (bi, fb, 0, 0))] * 4,
        out_shape=[cand_shape] * 4,
        scratch_shapes=[pltpu.VMEM((NCLS, 2 * BF, d), jnp.float32)],
        compiler_params=pltpu.CompilerParams(
            dimension_semantics=("arbitrary", "arbitrary", "arbitrary"),
            vmem_limit_bytes=60 * 1024 * 1024,
        ),
    )(xr)

    fsel_a, asel_a, bsel_a = _sc_merge_builder(b, d)(val, fsel, re, im)

    cand_spec = pl.BlockSpec((1, K, d), lambda bi, l0: (bi, 0, 0))
    xhat = pl.pallas_call(
        _recon_kernel,
        grid=(b, NL0),
        in_specs=[cand_spec] * 3,
        out_specs=pl.BlockSpec((1, LC, d), lambda bi, l0: (bi, l0, 0)),
        out_shape=jax.ShapeDtypeStruct((b, l, d), jnp.float32),
        scratch_shapes=[pltpu.VMEM((K, d), jnp.int32),
                        pltpu.VMEM((K, d), jnp.float32),
                        pltpu.VMEM((K, d), jnp.float32),
                        pltpu.VMEM((K, LC, d), jnp.float32),
                        pltpu.VMEM((K, LC, d), jnp.float32)],
        compiler_params=pltpu.CompilerParams(
            dimension_semantics=("arbitrary", "arbitrary"),
            vmem_limit_bytes=60 * 1024 * 1024,
        ),
    )(fsel_a, asel_a, bsel_a)
    return xhat
